# baseline (speedup 1.0000x reference)
import functools

import jax
import jax.numpy as jnp
from jax.experimental import pallas as pl
from jax.experimental.pallas import tpu as pltpu

_CH = 64


def _lstm_body(x_hbm, wih0_ref, whh0_ref, b0_ref, wih1_ref, whh1_ref, b1_ref,
               wfc_ref, bfc_ref, out_ref, gx_ref, xchunk_ref, sem,
               *, T, B, H, NS):
    bf = jnp.bfloat16
    nch = B // _CH
    Bs = B // NS
    col = jax.lax.broadcasted_iota(jnp.int32, (1, 4 * H), 1)
    scl = jnp.where((col >= 2 * H) & (col < 3 * H), 1.0, 0.5)

    def chunk_copy(ch):
        return pltpu.make_async_copy(
            x_hbm.at[pl.ds(ch * _CH, _CH)], xchunk_ref.at[ch],
            sem.at[ch],
        )

    for ch in range(nch):
        chunk_copy(ch).start()

    wih0 = (wih0_ref[...] * scl).astype(bf)
    b0 = b0_ref[...] * scl

    def cell(pre, c):
        tt = jnp.tanh(pre)
        tf = tt[:, 1 * H:2 * H]
        ti = tt[:, 0 * H:1 * H]
        tg = tt[:, 2 * H:3 * H]
        to = tt[:, 3 * H:4 * H]
        c = 0.5 * ((c + tf * c) + (tg + ti * tg))
        th = jnp.tanh(c)
        h = 0.5 * (th + to * th)
        return h, c

    whh0 = (whh0_ref[...] * scl).astype(bf)
    wih1 = (wih1_ref[...] * scl).astype(bf)
    whh1 = (whh1_ref[...] * scl).astype(bf)
    b1 = b1_ref[...] * scl
    wfc = wfc_ref[...].astype(bf)
    bfc = bfc_ref[...]

    ncs = Bs // _CH
    for sp in range(NS):
        rows = pl.ds(sp * Bs, Bs)
        for ch in range(sp * ncs, (sp + 1) * ncs):
            chunk_copy(ch).wait()
            xt = jnp.swapaxes(xchunk_ref[ch], 0, 1).astype(bf)
            gx = jnp.dot(xt.reshape(T * _CH, x_hbm.shape[-1]), wih0,
                         preferred_element_type=jnp.float32) + b0
            gx_ref[:, pl.ds(ch * _CH, _CH), :] = gx.reshape(T, _CH, 4 * H)
        h0 = jnp.zeros((Bs, H), jnp.float32)
        c0 = h0
        h1 = h0
        c1 = h0
        for t in range(T):
            pre0 = gx_ref[t, rows, :] + jnp.dot(
                h0.astype(bf), whh0, preferred_element_type=jnp.float32
            )
            h0, c0 = cell(pre0, c0)
            pre1 = b1 + jnp.dot(
                h0.astype(bf), wih1, preferred_element_type=jnp.float32
            ) + jnp.dot(
                h1.astype(bf), whh1, preferred_element_type=jnp.float32
            )
            h1, c1 = cell(pre1, c1)

        out_ref[rows, :] = (
            jnp.dot(h1.astype(bf), wfc,
                    preferred_element_type=jnp.float32)
            + bfc
        )


@jax.jit
def _forward(x, w_ih_0, w_hh_0, b_0, w_ih_1, w_hh_1, b_1, w_fc, b_fc):
    B, T, I = x.shape
    H = w_hh_0.shape[0]
    C = w_fc.shape[1]
    Bp = ((B + _CH - 1) // _CH) * _CH
    if Bp != B:
        x = jnp.pad(x, ((0, Bp - B), (0, 0), (0, 0)))

    ns = 2 if Bp % (2 * _CH) == 0 and Bp >= 2 * _CH else 1
    body = functools.partial(_lstm_body, T=T, B=Bp, H=H, NS=ns)
    bcast = lambda shape: pl.BlockSpec(shape, lambda: (0,) * len(shape))
    out = pl.pallas_call(
        body,
        out_shape=jax.ShapeDtypeStruct((Bp, C), jnp.float32),
        grid=(),
        in_specs=[
            pl.BlockSpec(memory_space=pl.ANY),
            bcast((I, 4 * H)), bcast((H, 4 * H)), bcast((1, 4 * H)),
            bcast((H, 4 * H)), bcast((H, 4 * H)), bcast((1, 4 * H)),
            bcast((H, C)), bcast((1, C)),
        ],
        out_specs=bcast((Bp, C)),
        scratch_shapes=[
            pltpu.VMEM((T, Bp, 4 * H), jnp.float32),
            pltpu.VMEM((Bp // _CH, _CH, T, I), jnp.float32),
            pltpu.SemaphoreType.DMA((Bp // _CH,)),
        ],
    )(x, w_ih_0, w_hh_0, b_0, w_ih_1, w_hh_1, b_1, w_fc, b_fc)
    return out[:B]


def kernel(x, w_ih_0, w_hh_0, b_0, w_ih_1, w_hh_1, b_1, w_fc, b_fc):
    return _forward(x, w_ih_0, w_hh_0, b_0, w_ih_1, w_hh_1, b_1, w_fc, b_fc)


# speedup vs baseline: 1.1093x; 1.1093x over previous
import functools

import jax
import jax.numpy as jnp
from jax.experimental import pallas as pl
from jax.experimental.pallas import tpu as pltpu

_CH = 64
_DEPTH = 3


def _lstm_body(x_hbm, wih0_ref, whh0_ref, b0_ref, wih1_ref, whh1_ref, b1_ref,
               wfc_ref, bfc_ref, out_ref, gx_ref, xchunk_ref, sem,
               *, T, B, H):
    bf = jnp.bfloat16
    nch = B // _CH
    col = jax.lax.broadcasted_iota(jnp.int32, (1, 4 * H), 1)
    scl = jnp.where((col >= 2 * H) & (col < 3 * H), 1.0, 0.5)

    def chunk_copy(ch):
        return pltpu.make_async_copy(
            x_hbm.at[pl.ds(ch * _CH, _CH)], xchunk_ref.at[ch % _DEPTH],
            sem.at[ch % _DEPTH],
        )

    for ch in range(min(_DEPTH, nch)):
        chunk_copy(ch).start()

    wih0 = (wih0_ref[...] * scl).astype(bf)
    b0 = b0_ref[...] * scl

    for ch in range(nch):
        slot = ch % _DEPTH
        chunk_copy(ch).wait()
        xt = jnp.swapaxes(xchunk_ref[slot], 0, 1).astype(bf)
        gx = jnp.dot(xt.reshape(T * _CH, x_hbm.shape[-1]), wih0,
                     preferred_element_type=jnp.float32) + b0
        gx_ref[:, pl.ds(ch * _CH, _CH), :] = gx.reshape(T, _CH, 4 * H)
        if ch + _DEPTH < nch:
            chunk_copy(ch + _DEPTH).start()

    def cell(pre, c):
        tt = jnp.tanh(pre)
        tf = tt[:, 1 * H:2 * H]
        ti = tt[:, 0 * H:1 * H]
        tg = tt[:, 2 * H:3 * H]
        to = tt[:, 3 * H:4 * H]
        c = 0.5 * ((c + tf * c) + (tg + ti * tg))
        th = jnp.tanh(c)
        h = 0.5 * (th + to * th)
        return h, c

    whh0 = (whh0_ref[...] * scl).astype(bf)
    wih1 = (wih1_ref[...] * scl).astype(bf)
    whh1 = (whh1_ref[...] * scl).astype(bf)
    b1 = b1_ref[...] * scl
    h0 = jnp.zeros((B, H), jnp.float32)
    c0 = h0
    h1 = h0
    c1 = h0
    for t in range(T):
        pre0 = gx_ref[t] + jnp.dot(
            h0.astype(bf), whh0, preferred_element_type=jnp.float32
        )
        h0, c0 = cell(pre0, c0)
        pre1 = b1 + jnp.dot(
            h0.astype(bf), wih1, preferred_element_type=jnp.float32
        ) + jnp.dot(
            h1.astype(bf), whh1, preferred_element_type=jnp.float32
        )
        h1, c1 = cell(pre1, c1)

    out_ref[...] = (
        jnp.dot(h1.astype(bf), wfc_ref[...].astype(bf),
                preferred_element_type=jnp.float32)
        + bfc_ref[...]
    )


@jax.jit
def _forward(x, w_ih_0, w_hh_0, b_0, w_ih_1, w_hh_1, b_1, w_fc, b_fc):
    B, T, I = x.shape
    H = w_hh_0.shape[0]
    C = w_fc.shape[1]
    Bp = ((B + _CH - 1) // _CH) * _CH
    if Bp != B:
        x = jnp.pad(x, ((0, Bp - B), (0, 0), (0, 0)))

    body = functools.partial(_lstm_body, T=T, B=Bp, H=H)
    bcast = lambda shape: pl.BlockSpec(shape, lambda: (0,) * len(shape))
    out = pl.pallas_call(
        body,
        out_shape=jax.ShapeDtypeStruct((Bp, C), jnp.float32),
        grid=(),
        in_specs=[
            pl.BlockSpec(memory_space=pl.ANY),
            bcast((I, 4 * H)), bcast((H, 4 * H)), bcast((1, 4 * H)),
            bcast((H, 4 * H)), bcast((H, 4 * H)), bcast((1, 4 * H)),
            bcast((H, C)), bcast((1, C)),
        ],
        out_specs=bcast((Bp, C)),
        scratch_shapes=[
            pltpu.VMEM((T, Bp, 4 * H), jnp.float32),
            pltpu.VMEM((_DEPTH, _CH, T, I), jnp.float32),
            pltpu.SemaphoreType.DMA((_DEPTH,)),
        ],
    )(x, w_ih_0, w_hh_0, b_0, w_ih_1, w_hh_1, b_1, w_fc, b_fc)
    return out[:B]


def kernel(x, w_ih_0, w_hh_0, b_0, w_ih_1, w_hh_1, b_1, w_fc, b_fc):
    return _forward(x, w_ih_0, w_hh_0, b_0, w_ih_1, w_hh_1, b_1, w_fc, b_fc)
